# Initial kernel scaffold; baseline (speedup 1.0000x reference)
import jax
import jax.numpy as jnp
from jax.experimental import pallas as pl


def kernel(waveform):
    raise NotImplementedError("write your pallas kernel here")


# speedup vs baseline: 404.2363x; 404.2363x over previous
import functools

import numpy as np
import jax
import jax.numpy as jnp
from jax import lax
from jax.experimental import pallas as pl
from jax.experimental.pallas import tpu as pltpu
from jax.experimental.pallas import tpu_sc as plsc

SR = 22050
N_FFT = 2048
HOP = 512
REF_POWER = 1e-6

NT = 5999
NF = 88
T_FRAMES = 2584
T_PAD = 2592
F_BINS = 1025
F_PAD = 1152

BT = 432
BF = 128

def _edges():
    ratio = 1.059463094
    lowest = 27.5
    hz = [lowest * ratio ** i for i in range(89)]
    fe = np.array([(x + y) / 2 for x, y in zip([lowest / ratio] + hz, hz)],
                  dtype=np.float64)
    te = np.arange(0.0, 60.0, 0.01)
    return te.astype(np.float32), fe.astype(np.float32)

_TE_F32, _FE_F32 = _edges()
TE_LEN = 6000
ED_PAD = 6144
_EDGE_TABLE = np.zeros((ED_PAD,), np.float32)
_EDGE_TABLE[:TE_LEN] = _TE_F32
_EDGE_TABLE[TE_LEN:TE_LEN + 89] = _FE_F32

_TE0 = np.float32(_TE_F32[0])
_TEL = np.float32(_TE_F32[-1])
_FE0 = np.float32(_FE_F32[0])
_FEL = np.float32(_FE_F32[-1])
_LOG2A = np.float64(np.log2(np.float64(27.5 * (1.0 + 1.059463094) / (2.0 * 1.059463094))))

NBINS = NT * NF
HPAD = 527936

def _trig():
    n = np.arange(N_FFT, dtype=np.float64)[:, None]
    k = np.arange(F_PAD, dtype=np.float64)[None, :]
    ang = 2.0 * np.pi * n * k / N_FFT
    cos = np.cos(ang)
    sin = np.sin(ang)
    cos[:, F_BINS:] = 0.0
    sin[:, F_BINS:] = 0.0
    return (cos.astype(np.float32).reshape(N_FFT, 9, BF),
            sin.astype(np.float32).reshape(N_FFT, 9, BF))

_COS_NP, _SIN_NP = _trig()

_HOP_SR = np.float32(512.0 / 22050.0)
_SR_NFFT = np.float32(11025.0 / 1024.0)
_FREQ_C = np.float32(0.5 * 22050.0 / np.pi)
_SR_F = np.float32(22050.0)


def _tc_body(c0, c1, c2, c3, w, t_out, f_out, w_out):
    i = pl.program_id(1)
    j = pl.program_id(0)
    hp = jax.lax.Precision.HIGHEST
    acc = jnp.dot(c0[...], w[pl.ds(0, 512), :], precision=hp, preferred_element_type=jnp.float32)
    acc += jnp.dot(c1[...], w[pl.ds(512, 512), :], precision=hp, preferred_element_type=jnp.float32)
    acc += jnp.dot(c2[...], w[pl.ds(1024, 512), :], precision=hp, preferred_element_type=jnp.float32)
    acc += jnp.dot(c3[...], w[pl.ds(1536, 512), :], precision=hp, preferred_element_type=jnp.float32)

    re_h = acc[:, 0 * BF:1 * BF]
    im_h = acc[:, 1 * BF:2 * BF]
    re_dh = acc[:, 2 * BF:3 * BF]
    im_dh = acc[:, 3 * BF:4 * BF]
    re_th = acc[:, 4 * BF:5 * BF]
    im_th = acc[:, 5 * BF:6 * BF]

    power = re_h * re_h + im_h * im_h
    mags = jnp.sqrt(power)
    bad = power < np.float32(REF_POWER)

    freq_corr = -((im_dh * re_h - re_dh * im_h) / power) * _FREQ_C
    time_corr = ((re_th * re_h + im_th * im_h) / power) / _SR_F

    rows = i * BT + lax.broadcasted_iota(jnp.int32, (BT, 1), 0)
    ft = rows.astype(jnp.float32) * _HOP_SR
    cols = j * BF + lax.broadcasted_iota(jnp.int32, (1, BF), 1)
    bf = cols.astype(jnp.float32) * _SR_NFFT

    times = jnp.where(bad, jnp.broadcast_to(ft, power.shape), ft + time_corr)
    freqs = jnp.where(bad, jnp.broadcast_to(bf, power.shape), bf + freq_corr)
    valid = ((times >= _TE0) & (times <= _TEL)
             & (freqs >= _FE0) & (freqs <= _FEL))
    wgt = jnp.where(valid, mags, np.float32(0.0))

    t_out[...] = times
    f_out[...] = freqs
    w_out[...] = wgt


def _run_tc(c0, c1, c2, c3, wmat):
    spec_c = pl.BlockSpec((BT, 512), lambda j, i: (i, 0))
    spec_w = pl.BlockSpec((N_FFT, 6 * BF), lambda j, i: (0, j))
    spec_o = pl.BlockSpec((BT, BF), lambda j, i: (i, j))
    out_sh = jax.ShapeDtypeStruct((T_PAD, F_PAD), jnp.float32)
    return pl.pallas_call(
        _tc_body,
        grid=(F_PAD // BF, T_PAD // BT),
        in_specs=[spec_c, spec_c, spec_c, spec_c, spec_w],
        out_specs=[spec_o, spec_o, spec_o],
        out_shape=[out_sh, out_sh, out_sh],
    )(c0, c1, c2, c3, wmat)


N_ELEM = T_PAD * F_PAD
N_ROWS = N_ELEM // 128
CHUNK_R = 8
N_CHUNK = 92
ROWS_PT = CHUNK_R * N_CHUNK
N_ROWS_PAD = 32 * ROWS_PT


def _bin16(t16, f16, ed_v):
    c0 = jnp.clip(t16 * np.float32(100.0), np.float32(-10.0),
                  np.float32(6100.0)).astype(jnp.int32)
    bt = jnp.clip(c0 - 1, 0, TE_LEN - 3)
    e0 = plsc.load_gather(ed_v, [bt])
    e1 = plsc.load_gather(ed_v, [bt + 1])
    e2 = plsc.load_gather(ed_v, [bt + 2])
    cnt = ((t16 >= e0).astype(jnp.int32) + (t16 >= e1).astype(jnp.int32)
           + (t16 >= e2).astype(jnp.int32))
    ti = jnp.clip(bt + cnt - 1, 0, NT - 1)
    bits = plsc.bitcast(f16, jnp.int32)
    ex = (lax.shift_right_logical(bits, 23) & 255) - 127
    mant = (bits & 0x7FFFFF).astype(jnp.float32) * np.float32(2.0 ** -23)
    l2 = ex.astype(jnp.float32) + mant + mant * (np.float32(1.0) - mant) * np.float32(0.343)
    jf = (l2 - np.float32(_LOG2A)) * np.float32(12.0)
    j0 = jnp.clip(jf, np.float32(-10.0), np.float32(200.0)).astype(jnp.int32)
    j0 = jnp.where(f16 >= np.float32(1.0), j0, 0)
    bfq = jnp.clip(j0 - 1, 0, NF - 2) + TE_LEN
    g0 = plsc.load_gather(ed_v, [bfq])
    g1 = plsc.load_gather(ed_v, [bfq + 1])
    g2 = plsc.load_gather(ed_v, [bfq + 2])
    cf = ((f16 >= g0).astype(jnp.int32) + (f16 >= g1).astype(jnp.int32)
          + (f16 >= g2).astype(jnp.int32))
    fi = jnp.clip(bfq - TE_LEN + cf - 1, 0, NF - 1)
    return ti * NF + fi


def _sc_body(t_hbm, f_hbm, w_hbm, ed_hbm, z_hbm, out0_hbm, out1_hbm,
             ed_v, tv, fv, wv, iv, hist):
    c = lax.axis_index("c")
    s = lax.axis_index("s")
    wid = s * 2 + c
    pltpu.sync_copy(ed_hbm, ed_v)

    @pl.when(s == 0)
    def _():
        pltpu.sync_copy(z_hbm, hist)
    plsc.subcore_barrier()

    base_row = wid * ROWS_PT

    def chunk(g, carry):
        row0 = base_row + g * CHUNK_R
        pltpu.sync_copy(t_hbm.at[pl.ds(row0, CHUNK_R)], tv)
        pltpu.sync_copy(f_hbm.at[pl.ds(row0, CHUNK_R)], fv)
        pltpu.sync_copy(w_hbm.at[pl.ds(row0, CHUNK_R)], wv)
        for r in range(CHUNK_R):
            for v in range(8):
                sl = pl.ds(v * 16, 16)
                iv[r, sl] = _bin16(tv[r, sl], fv[r, sl], ed_v)
        for r in range(CHUNK_R):
            pltpu.sync_copy(wv.at[r], hist.at[iv.at[r]], add=True)
        return carry

    lax.fori_loop(0, N_CHUNK, chunk, 0)
    plsc.subcore_barrier()

    @pl.when((s == 0) & (c == 0))
    def _():
        pltpu.sync_copy(hist, out0_hbm)

    @pl.when((s == 0) & (c == 1))
    def _():
        pltpu.sync_copy(hist, out1_hbm)


@functools.lru_cache(maxsize=1)
def _get_sc_hist():
    mesh = plsc.VectorSubcoreMesh(core_axis_name="c", subcore_axis_name="s")
    return functools.partial(
        pl.kernel,
        mesh=mesh,
        compiler_params=pltpu.CompilerParams(needs_layout_passes=False),
        out_type=[jax.ShapeDtypeStruct((HPAD,), jnp.float32),
                  jax.ShapeDtypeStruct((HPAD,), jnp.float32)],
        scratch_types=[
            pltpu.VMEM((ED_PAD,), jnp.float32),
            pltpu.VMEM((CHUNK_R, 128), jnp.float32),
            pltpu.VMEM((CHUNK_R, 128), jnp.float32),
            pltpu.VMEM((CHUNK_R, 128), jnp.float32),
            pltpu.VMEM((CHUNK_R, 128), jnp.int32),
            pltpu.VMEM_SHARED((HPAD,), jnp.float32),
        ],
    )(_sc_body)


def kernel(waveform):
    pad = N_FFT // 2
    ypad = jnp.pad(waveform, (pad, pad))
    cgrid = ypad[: (T_FRAMES + 3) * HOP].reshape(T_FRAMES + 3, HOP)
    cs = [jnp.pad(cgrid[k:T_FRAMES + k], ((0, T_PAD - T_FRAMES), (0, 0)))
          for k in range(4)]

    n = jnp.arange(N_FFT)
    win = (0.5 - 0.5 * jnp.cos(2.0 * jnp.pi * n / N_FFT)).astype(jnp.float32)
    dwin = (jnp.roll(win, -1) - jnp.roll(win, 1)) * 0.5
    wtimes = (jnp.arange(N_FFT) + 0.5 - N_FFT // 2).astype(jnp.float32)
    twin = win * wtimes

    cosm = jnp.asarray(_COS_NP)
    sinm = jnp.asarray(_SIN_NP)
    wmat = jnp.stack(
        [win[:, None, None] * cosm, -(win[:, None, None] * sinm),
         dwin[:, None, None] * cosm, -(dwin[:, None, None] * sinm),
         twin[:, None, None] * cosm, -(twin[:, None, None] * sinm)],
        axis=2,
    ).reshape(N_FFT, 9 * 6 * BF)

    t, f, w = _run_tc(cs[0], cs[1], cs[2], cs[3], wmat)

    rpad = ((0, N_ROWS_PAD - N_ROWS), (0, 0))
    t2 = jnp.pad(t.reshape(N_ROWS, 128), rpad)
    f2 = jnp.pad(f.reshape(N_ROWS, 128), rpad)
    w2 = jnp.pad(w.reshape(N_ROWS, 128), rpad)
    ed = jnp.asarray(_EDGE_TABLE)
    z = jnp.zeros((HPAD,), jnp.float32)

    p0, p1 = _get_sc_hist()(t2, f2, w2, ed, z)
    return (p0[:NBINS] + p1[:NBINS]).reshape(NT, NF)


# speedup vs baseline: 505.7778x; 1.2512x over previous
import functools

import numpy as np
import jax
import jax.numpy as jnp
from jax import lax
from jax.experimental import pallas as pl
from jax.experimental.pallas import tpu as pltpu
from jax.experimental.pallas import tpu_sc as plsc

SR = 22050
N_FFT = 2048
HOP = 512
REF_POWER = 1e-6

NT = 5999
NF = 88
T_FRAMES = 2584
T_PAD = 2592
F_BINS = 1025
F_PAD = 1152

BT = 432
BF = 128

def _edges():
    ratio = 1.059463094
    lowest = 27.5
    hz = [lowest * ratio ** i for i in range(89)]
    fe = np.array([(x + y) / 2 for x, y in zip([lowest / ratio] + hz, hz)],
                  dtype=np.float64)
    te = np.arange(0.0, 60.0, 0.01)
    return te.astype(np.float32), fe.astype(np.float32)

_TE_F32, _FE_F32 = _edges()
TE_LEN = 6000
ED_PAD = 6144
_EDGE_TABLE = np.zeros((ED_PAD,), np.float32)
_EDGE_TABLE[:TE_LEN] = _TE_F32
_EDGE_TABLE[TE_LEN:TE_LEN + 89] = _FE_F32

_TE0 = np.float32(_TE_F32[0])
_TEL = np.float32(_TE_F32[-1])
_FE0 = np.float32(_FE_F32[0])
_FEL = np.float32(_FE_F32[-1])
_LOG2A = np.float64(np.log2(np.float64(27.5 * (1.0 + 1.059463094) / (2.0 * 1.059463094))))

NBINS = NT * NF
HPAD = 527936

def _trig():
    n = np.arange(N_FFT, dtype=np.float64)[:, None]
    k = np.arange(F_PAD, dtype=np.float64)[None, :]
    ang = 2.0 * np.pi * n * k / N_FFT
    cos = np.cos(ang)
    sin = np.sin(ang)
    cos[:, F_BINS:] = 0.0
    sin[:, F_BINS:] = 0.0
    return (cos.astype(np.float32).reshape(N_FFT, 9, BF),
            sin.astype(np.float32).reshape(N_FFT, 9, BF))

_COS_NP, _SIN_NP = _trig()

_HOP_SR = np.float32(512.0 / 22050.0)
_SR_NFFT = np.float32(11025.0 / 1024.0)
_FREQ_C = np.float32(0.5 * 22050.0 / np.pi)
_SR_F = np.float32(22050.0)


def _dot3(a, b):
    ah = a.astype(jnp.bfloat16)
    al = (a - ah.astype(jnp.float32)).astype(jnp.bfloat16)
    bh = b.astype(jnp.bfloat16)
    bl = (b - bh.astype(jnp.float32)).astype(jnp.bfloat16)
    d = lambda x, y: jnp.dot(x, y, preferred_element_type=jnp.float32)
    return d(ah, bh) + (d(ah, bl) + d(al, bh))


def _tc_body(c0, c1, c2, c3, w, t_out, f_out, w_out):
    i = pl.program_id(1)
    j = pl.program_id(0)
    acc = _dot3(c0[...], w[pl.ds(0, 512), :])
    acc += _dot3(c1[...], w[pl.ds(512, 512), :])
    acc += _dot3(c2[...], w[pl.ds(1024, 512), :])
    acc += _dot3(c3[...], w[pl.ds(1536, 512), :])

    re_h = acc[:, 0 * BF:1 * BF]
    im_h = acc[:, 1 * BF:2 * BF]
    re_dh = acc[:, 2 * BF:3 * BF]
    im_dh = acc[:, 3 * BF:4 * BF]
    re_th = acc[:, 4 * BF:5 * BF]
    im_th = acc[:, 5 * BF:6 * BF]

    power = re_h * re_h + im_h * im_h
    mags = jnp.sqrt(power)
    bad = power < np.float32(REF_POWER)

    freq_corr = -((im_dh * re_h - re_dh * im_h) / power) * _FREQ_C
    time_corr = ((re_th * re_h + im_th * im_h) / power) / _SR_F

    rows = i * BT + lax.broadcasted_iota(jnp.int32, (BT, 1), 0)
    ft = rows.astype(jnp.float32) * _HOP_SR
    cols = j * BF + lax.broadcasted_iota(jnp.int32, (1, BF), 1)
    bf = cols.astype(jnp.float32) * _SR_NFFT

    times = jnp.where(bad, jnp.broadcast_to(ft, power.shape), ft + time_corr)
    freqs = jnp.where(bad, jnp.broadcast_to(bf, power.shape), bf + freq_corr)
    valid = ((times >= _TE0) & (times <= _TEL)
             & (freqs >= _FE0) & (freqs <= _FEL))
    wgt = jnp.where(valid, mags, np.float32(0.0))

    t_out[...] = times
    f_out[...] = freqs
    w_out[...] = wgt


def _run_tc(c0, c1, c2, c3, wmat):
    spec_c = pl.BlockSpec((BT, 512), lambda j, i: (i, 0))
    spec_w = pl.BlockSpec((N_FFT, 6 * BF), lambda j, i: (0, j))
    spec_o = pl.BlockSpec((BT, BF), lambda j, i: (i, j))
    out_sh = jax.ShapeDtypeStruct((T_PAD, F_PAD), jnp.float32)
    return pl.pallas_call(
        _tc_body,
        grid=(F_PAD // BF, T_PAD // BT),
        in_specs=[spec_c, spec_c, spec_c, spec_c, spec_w],
        out_specs=[spec_o, spec_o, spec_o],
        out_shape=[out_sh, out_sh, out_sh],
    )(c0, c1, c2, c3, wmat)


N_ELEM = T_PAD * F_PAD
N_ROWS = N_ELEM // 128
CHUNK_R = 8
N_CHUNK = 92
ROWS_PT = CHUNK_R * N_CHUNK
N_ROWS_PAD = 32 * ROWS_PT


def _bin16(t16, f16, ed_v):
    c0 = jnp.clip(t16 * np.float32(100.0), np.float32(-10.0),
                  np.float32(6100.0)).astype(jnp.int32)
    bt = jnp.clip(c0 - 1, 0, TE_LEN - 3)
    e0 = plsc.load_gather(ed_v, [bt])
    e1 = plsc.load_gather(ed_v, [bt + 1])
    e2 = plsc.load_gather(ed_v, [bt + 2])
    cnt = ((t16 >= e0).astype(jnp.int32) + (t16 >= e1).astype(jnp.int32)
           + (t16 >= e2).astype(jnp.int32))
    ti = jnp.clip(bt + cnt - 1, 0, NT - 1)
    bits = plsc.bitcast(f16, jnp.int32)
    ex = (lax.shift_right_logical(bits, 23) & 255) - 127
    mant = (bits & 0x7FFFFF).astype(jnp.float32) * np.float32(2.0 ** -23)
    l2 = ex.astype(jnp.float32) + mant + mant * (np.float32(1.0) - mant) * np.float32(0.343)
    jf = (l2 - np.float32(_LOG2A)) * np.float32(12.0)
    j0 = jnp.clip(jf, np.float32(-10.0), np.float32(200.0)).astype(jnp.int32)
    j0 = jnp.where(f16 >= np.float32(1.0), j0, 0)
    bfq = jnp.clip(j0 - 1, 0, NF - 2) + TE_LEN
    g0 = plsc.load_gather(ed_v, [bfq])
    g1 = plsc.load_gather(ed_v, [bfq + 1])
    g2 = plsc.load_gather(ed_v, [bfq + 2])
    cf = ((f16 >= g0).astype(jnp.int32) + (f16 >= g1).astype(jnp.int32)
          + (f16 >= g2).astype(jnp.int32))
    fi = jnp.clip(bfq - TE_LEN + cf - 1, 0, NF - 1)
    return ti * NF + fi


def _sc_body(t_hbm, f_hbm, w_hbm, ed_hbm, z_hbm, out0_hbm, out1_hbm,
             ed_v, tv, fv, wv, iv, hist):
    c = lax.axis_index("c")
    s = lax.axis_index("s")
    wid = s * 2 + c
    pltpu.sync_copy(ed_hbm, ed_v)

    @pl.when(s == 0)
    def _():
        pltpu.sync_copy(z_hbm, hist)
    plsc.subcore_barrier()

    base_row = wid * ROWS_PT

    def chunk(g, carry):
        row0 = base_row + g * CHUNK_R
        pltpu.sync_copy(t_hbm.at[pl.ds(row0, CHUNK_R)], tv)
        pltpu.sync_copy(f_hbm.at[pl.ds(row0, CHUNK_R)], fv)
        pltpu.sync_copy(w_hbm.at[pl.ds(row0, CHUNK_R)], wv)
        for r in range(CHUNK_R):
            for v in range(8):
                sl = pl.ds(v * 16, 16)
                iv[r, sl] = _bin16(tv[r, sl], fv[r, sl], ed_v)
        for r in range(CHUNK_R):
            pltpu.sync_copy(wv.at[r], hist.at[iv.at[r]], add=True)
        return carry

    lax.fori_loop(0, N_CHUNK, chunk, 0)
    plsc.subcore_barrier()

    @pl.when((s == 0) & (c == 0))
    def _():
        pltpu.sync_copy(hist, out0_hbm)

    @pl.when((s == 0) & (c == 1))
    def _():
        pltpu.sync_copy(hist, out1_hbm)


@functools.lru_cache(maxsize=1)
def _get_sc_hist():
    mesh = plsc.VectorSubcoreMesh(core_axis_name="c", subcore_axis_name="s")
    return functools.partial(
        pl.kernel,
        mesh=mesh,
        compiler_params=pltpu.CompilerParams(needs_layout_passes=False),
        out_type=[jax.ShapeDtypeStruct((HPAD,), jnp.float32),
                  jax.ShapeDtypeStruct((HPAD,), jnp.float32)],
        scratch_types=[
            pltpu.VMEM((ED_PAD,), jnp.float32),
            pltpu.VMEM((CHUNK_R, 128), jnp.float32),
            pltpu.VMEM((CHUNK_R, 128), jnp.float32),
            pltpu.VMEM((CHUNK_R, 128), jnp.float32),
            pltpu.VMEM((CHUNK_R, 128), jnp.int32),
            pltpu.VMEM_SHARED((HPAD,), jnp.float32),
        ],
    )(_sc_body)


def kernel(waveform):
    pad = N_FFT // 2
    ypad = jnp.pad(waveform, (pad, pad))
    cgrid = ypad[: (T_FRAMES + 3) * HOP].reshape(T_FRAMES + 3, HOP)
    cs = [jnp.pad(cgrid[k:T_FRAMES + k], ((0, T_PAD - T_FRAMES), (0, 0)))
          for k in range(4)]

    n = jnp.arange(N_FFT)
    win = (0.5 - 0.5 * jnp.cos(2.0 * jnp.pi * n / N_FFT)).astype(jnp.float32)
    dwin = (jnp.roll(win, -1) - jnp.roll(win, 1)) * 0.5
    wtimes = (jnp.arange(N_FFT) + 0.5 - N_FFT // 2).astype(jnp.float32)
    twin = win * wtimes

    cosm = jnp.asarray(_COS_NP)
    sinm = jnp.asarray(_SIN_NP)
    wmat = jnp.stack(
        [win[:, None, None] * cosm, -(win[:, None, None] * sinm),
         dwin[:, None, None] * cosm, -(dwin[:, None, None] * sinm),
         twin[:, None, None] * cosm, -(twin[:, None, None] * sinm)],
        axis=2,
    ).reshape(N_FFT, 9 * 6 * BF)

    t, f, w = _run_tc(cs[0], cs[1], cs[2], cs[3], wmat)

    rpad = ((0, N_ROWS_PAD - N_ROWS), (0, 0))
    t2 = jnp.pad(t.reshape(N_ROWS, 128), rpad)
    f2 = jnp.pad(f.reshape(N_ROWS, 128), rpad)
    w2 = jnp.pad(w.reshape(N_ROWS, 128), rpad)
    ed = jnp.asarray(_EDGE_TABLE)
    z = jnp.zeros((HPAD,), jnp.float32)

    p0, p1 = _get_sc_hist()(t2, f2, w2, ed, z)
    return (p0[:NBINS] + p1[:NBINS]).reshape(NT, NF)


# speedup vs baseline: 756.9109x; 1.4965x over previous
import functools

import numpy as np
import jax
import jax.numpy as jnp
from jax import lax
from jax.experimental import pallas as pl
from jax.experimental.pallas import tpu as pltpu
from jax.experimental.pallas import tpu_sc as plsc

SR = 22050
N_FFT = 2048
HOP = 512
REF_POWER = 1e-6

NT = 5999
NF = 88
T_FRAMES = 2584
T_PAD = 2592
F_BINS = 1025
F_PAD = 1152

BT = 432
BF = 128

def _edges():
    ratio = 1.059463094
    lowest = 27.5
    hz = [lowest * ratio ** i for i in range(89)]
    fe = np.array([(x + y) / 2 for x, y in zip([lowest / ratio] + hz, hz)],
                  dtype=np.float64)
    te = np.arange(0.0, 60.0, 0.01)
    return te.astype(np.float32), fe.astype(np.float32)

_TE_F32, _FE_F32 = _edges()
TE_LEN = 6000
ED_PAD = 6144
_EDGE_TABLE = np.zeros((ED_PAD,), np.float32)
_EDGE_TABLE[:TE_LEN] = _TE_F32
_EDGE_TABLE[TE_LEN:TE_LEN + 89] = _FE_F32

_TE0 = np.float32(_TE_F32[0])
_TEL = np.float32(_TE_F32[-1])
_FE0 = np.float32(_FE_F32[0])
_FEL = np.float32(_FE_F32[-1])
_LOG2A = np.float64(np.log2(np.float64(27.5 * (1.0 + 1.059463094) / (2.0 * 1.059463094))))

NBINS = NT * NF
HPAD = 527936

def _trig():
    n = np.arange(N_FFT, dtype=np.float64)[:, None]
    k = np.arange(F_PAD, dtype=np.float64)[None, :]
    ang = 2.0 * np.pi * n * k / N_FFT
    cos = np.cos(ang)
    sin = np.sin(ang)
    cos[:, F_BINS:] = 0.0
    sin[:, F_BINS:] = 0.0
    return (cos.astype(np.float32).reshape(N_FFT, 9, BF),
            sin.astype(np.float32).reshape(N_FFT, 9, BF))

_COS_NP, _SIN_NP = _trig()

_HOP_SR = np.float32(512.0 / 22050.0)
_SR_NFFT = np.float32(11025.0 / 1024.0)
_FREQ_C = np.float32(0.5 * 22050.0 / np.pi)
_SR_F = np.float32(22050.0)


def _dot3(a, b):
    ah = a.astype(jnp.bfloat16)
    al = (a - ah.astype(jnp.float32)).astype(jnp.bfloat16)
    bh = b.astype(jnp.bfloat16)
    bl = (b - bh.astype(jnp.float32)).astype(jnp.bfloat16)
    d = lambda x, y: jnp.dot(x, y, preferred_element_type=jnp.float32)
    return d(ah, bh) + (d(ah, bl) + d(al, bh))


def _tc_body(c0, c1, c2, c3, w, t_out, f_out, w_out):
    i = pl.program_id(1)
    j = pl.program_id(0)
    acc = _dot3(c0[...], w[pl.ds(0, 512), :])
    acc += _dot3(c1[...], w[pl.ds(512, 512), :])
    acc += _dot3(c2[...], w[pl.ds(1024, 512), :])
    acc += _dot3(c3[...], w[pl.ds(1536, 512), :])

    re_h = acc[:, 0 * BF:1 * BF]
    im_h = acc[:, 1 * BF:2 * BF]
    re_dh = acc[:, 2 * BF:3 * BF]
    im_dh = acc[:, 3 * BF:4 * BF]
    re_th = acc[:, 4 * BF:5 * BF]
    im_th = acc[:, 5 * BF:6 * BF]

    power = re_h * re_h + im_h * im_h
    mags = jnp.sqrt(power)
    bad = power < np.float32(REF_POWER)

    freq_corr = -((im_dh * re_h - re_dh * im_h) / power) * _FREQ_C
    time_corr = ((re_th * re_h + im_th * im_h) / power) / _SR_F

    rows = i * BT + lax.broadcasted_iota(jnp.int32, (BT, 1), 0)
    ft = rows.astype(jnp.float32) * _HOP_SR
    cols = j * BF + lax.broadcasted_iota(jnp.int32, (1, BF), 1)
    bf = cols.astype(jnp.float32) * _SR_NFFT

    times = jnp.where(bad, jnp.broadcast_to(ft, power.shape), ft + time_corr)
    freqs = jnp.where(bad, jnp.broadcast_to(bf, power.shape), bf + freq_corr)
    valid = ((times >= _TE0) & (times <= _TEL)
             & (freqs >= _FE0) & (freqs <= _FEL))
    wgt = jnp.where(valid, mags, np.float32(0.0))

    t_out[...] = times
    f_out[...] = freqs
    w_out[...] = wgt


def _run_tc(c0, c1, c2, c3, wmat):
    spec_c = pl.BlockSpec((BT, 512), lambda j, i: (i, 0))
    spec_w = pl.BlockSpec((N_FFT, 6 * BF), lambda j, i: (0, j))
    spec_o = pl.BlockSpec((BT, BF), lambda j, i: (i, j))
    out_sh = jax.ShapeDtypeStruct((T_PAD, F_PAD), jnp.float32)
    return pl.pallas_call(
        _tc_body,
        grid=(F_PAD // BF, T_PAD // BT),
        in_specs=[spec_c, spec_c, spec_c, spec_c, spec_w],
        out_specs=[spec_o, spec_o, spec_o],
        out_shape=[out_sh, out_sh, out_sh],
    )(c0, c1, c2, c3, wmat)


N_ELEM = T_PAD * F_PAD
N_ROWS = N_ELEM // 128
CHUNK_R = 8
N_CHUNK = 92
ROWS_PT = CHUNK_R * N_CHUNK
N_ROWS_PAD = 32 * ROWS_PT


def _bin16(t16, f16, ed_v):
    c0 = jnp.clip(t16 * np.float32(100.0), np.float32(-10.0),
                  np.float32(6100.0)).astype(jnp.int32)
    bt = jnp.clip(c0 - 1, 0, TE_LEN - 3)
    e0 = plsc.load_gather(ed_v, [bt])
    e1 = plsc.load_gather(ed_v, [bt + 1])
    e2 = plsc.load_gather(ed_v, [bt + 2])
    cnt = ((t16 >= e0).astype(jnp.int32) + (t16 >= e1).astype(jnp.int32)
           + (t16 >= e2).astype(jnp.int32))
    ti = jnp.clip(bt + cnt - 1, 0, NT - 1)
    bits = plsc.bitcast(f16, jnp.int32)
    ex = (lax.shift_right_logical(bits, 23) & 255) - 127
    mant = (bits & 0x7FFFFF).astype(jnp.float32) * np.float32(2.0 ** -23)
    l2 = ex.astype(jnp.float32) + mant + mant * (np.float32(1.0) - mant) * np.float32(0.343)
    jf = (l2 - np.float32(_LOG2A)) * np.float32(12.0)
    j0 = jnp.clip(jf, np.float32(-10.0), np.float32(200.0)).astype(jnp.int32)
    j0 = jnp.where(f16 >= np.float32(1.0), j0, 0)
    bfq = jnp.clip(j0 - 1, 0, NF - 2) + TE_LEN
    g0 = plsc.load_gather(ed_v, [bfq])
    g1 = plsc.load_gather(ed_v, [bfq + 1])
    g2 = plsc.load_gather(ed_v, [bfq + 2])
    cf = ((f16 >= g0).astype(jnp.int32) + (f16 >= g1).astype(jnp.int32)
          + (f16 >= g2).astype(jnp.int32))
    fi = jnp.clip(bfq - TE_LEN + cf - 1, 0, NF - 1)
    return ti * NF + fi


def _sc_body(t_hbm, f_hbm, w_hbm, ed_hbm, z_hbm, out0_hbm, out1_hbm,
             ed_v, tv0, fv0, wv0, iv0, tv1, fv1, wv1, iv1, hist,
             lsem0, lsem1, ssem0, ssem1):
    c = lax.axis_index("c")
    s = lax.axis_index("s")
    wid = s * 2 + c
    pltpu.sync_copy(ed_hbm, ed_v)

    @pl.when(s == 0)
    def _():
        pltpu.sync_copy(z_hbm, hist)
    plsc.subcore_barrier()

    base_row = wid * ROWS_PT

    def loads(row0, tvb, fvb, wvb, sem):
        pltpu.async_copy(t_hbm.at[pl.ds(row0, CHUNK_R)], tvb, sem)
        pltpu.async_copy(f_hbm.at[pl.ds(row0, CHUNK_R)], fvb, sem)
        pltpu.async_copy(w_hbm.at[pl.ds(row0, CHUNK_R)], wvb, sem)

    def wait_loads(tvb, fvb, wvb, sem):
        pltpu.make_async_copy(t_hbm.at[pl.ds(0, CHUNK_R)], tvb, sem).wait()
        pltpu.make_async_copy(f_hbm.at[pl.ds(0, CHUNK_R)], fvb, sem).wait()
        pltpu.make_async_copy(w_hbm.at[pl.ds(0, CHUNK_R)], wvb, sem).wait()

    def compute(tvb, fvb, ivb):
        def row(r, carry):
            for v in range(8):
                sl = pl.ds(v * 16, 16)
                ivb[r, sl] = _bin16(tvb[r, sl], fvb[r, sl], ed_v)
            return carry
        lax.fori_loop(0, CHUNK_R, row, 0)

    def scatter(wvb, ivb, sem):
        for r in range(CHUNK_R):
            pltpu.async_copy(wvb.at[r], hist.at[ivb.at[r]], sem, add=True)

    def wait_scatter(wvb, ivb, sem):
        for r in range(CHUNK_R):
            pltpu.make_async_copy(wvb.at[r], hist.at[ivb.at[r]], sem).wait()

    loads(base_row, tv0, fv0, wv0, lsem0)
    loads(base_row + CHUNK_R, tv1, fv1, wv1, lsem1)

    def body(k, carry):
        row_a = base_row + 2 * k * CHUNK_R
        wait_loads(tv0, fv0, wv0, lsem0)
        compute(tv0, fv0, iv0)
        scatter(wv0, iv0, ssem0)
        wait_loads(tv1, fv1, wv1, lsem1)
        compute(tv1, fv1, iv1)
        scatter(wv1, iv1, ssem1)

        @pl.when(k < N_CHUNK // 2 - 1)
        def _():
            wait_scatter(wv0, iv0, ssem0)
            loads(row_a + 2 * CHUNK_R, tv0, fv0, wv0, lsem0)
            wait_scatter(wv1, iv1, ssem1)
            loads(row_a + 3 * CHUNK_R, tv1, fv1, wv1, lsem1)
        return carry

    lax.fori_loop(0, N_CHUNK // 2, body, 0)
    wait_scatter(wv0, iv0, ssem0)
    wait_scatter(wv1, iv1, ssem1)
    plsc.subcore_barrier()

    @pl.when((s == 0) & (c == 0))
    def _():
        pltpu.sync_copy(hist, out0_hbm)

    @pl.when((s == 0) & (c == 1))
    def _():
        pltpu.sync_copy(hist, out1_hbm)


@functools.lru_cache(maxsize=1)
def _get_sc_hist():
    mesh = plsc.VectorSubcoreMesh(core_axis_name="c", subcore_axis_name="s")
    return functools.partial(
        pl.kernel,
        mesh=mesh,
        compiler_params=pltpu.CompilerParams(needs_layout_passes=False),
        out_type=[jax.ShapeDtypeStruct((HPAD,), jnp.float32),
                  jax.ShapeDtypeStruct((HPAD,), jnp.float32)],
        scratch_types=[
            pltpu.VMEM((ED_PAD,), jnp.float32),
            pltpu.VMEM((CHUNK_R, 128), jnp.float32),
            pltpu.VMEM((CHUNK_R, 128), jnp.float32),
            pltpu.VMEM((CHUNK_R, 128), jnp.float32),
            pltpu.VMEM((CHUNK_R, 128), jnp.int32),
            pltpu.VMEM((CHUNK_R, 128), jnp.float32),
            pltpu.VMEM((CHUNK_R, 128), jnp.float32),
            pltpu.VMEM((CHUNK_R, 128), jnp.float32),
            pltpu.VMEM((CHUNK_R, 128), jnp.int32),
            pltpu.VMEM_SHARED((HPAD,), jnp.float32),
            pltpu.SemaphoreType.DMA,
            pltpu.SemaphoreType.DMA,
            pltpu.SemaphoreType.DMA,
            pltpu.SemaphoreType.DMA,
        ],
    )(_sc_body)


def kernel(waveform):
    pad = N_FFT // 2
    ypad = jnp.pad(waveform, (pad, pad))
    cgrid = ypad[: (T_FRAMES + 3) * HOP].reshape(T_FRAMES + 3, HOP)
    cs = [jnp.pad(cgrid[k:T_FRAMES + k], ((0, T_PAD - T_FRAMES), (0, 0)))
          for k in range(4)]

    n = jnp.arange(N_FFT)
    win = (0.5 - 0.5 * jnp.cos(2.0 * jnp.pi * n / N_FFT)).astype(jnp.float32)
    dwin = (jnp.roll(win, -1) - jnp.roll(win, 1)) * 0.5
    wtimes = (jnp.arange(N_FFT) + 0.5 - N_FFT // 2).astype(jnp.float32)
    twin = win * wtimes

    cosm = jnp.asarray(_COS_NP)
    sinm = jnp.asarray(_SIN_NP)
    wmat = jnp.stack(
        [win[:, None, None] * cosm, -(win[:, None, None] * sinm),
         dwin[:, None, None] * cosm, -(dwin[:, None, None] * sinm),
         twin[:, None, None] * cosm, -(twin[:, None, None] * sinm)],
        axis=2,
    ).reshape(N_FFT, 9 * 6 * BF)

    t, f, w = _run_tc(cs[0], cs[1], cs[2], cs[3], wmat)

    rpad = ((0, N_ROWS_PAD - N_ROWS), (0, 0))
    t2 = jnp.pad(t.reshape(N_ROWS, 128), rpad)
    f2 = jnp.pad(f.reshape(N_ROWS, 128), rpad)
    w2 = jnp.pad(w.reshape(N_ROWS, 128), rpad)
    ed = jnp.asarray(_EDGE_TABLE)
    z = jnp.zeros((HPAD,), jnp.float32)

    p0, p1 = _get_sc_hist()(t2, f2, w2, ed, z)
    return (p0[:NBINS] + p1[:NBINS]).reshape(NT, NF)


# speedup vs baseline: 960.1054x; 1.2685x over previous
import functools

import numpy as np
import jax
import jax.numpy as jnp
from jax import lax
from jax.experimental import pallas as pl
from jax.experimental.pallas import tpu as pltpu
from jax.experimental.pallas import tpu_sc as plsc

SR = 22050
N_FFT = 2048
HOP = 512
REF_POWER = 1e-6

NT = 5999
NF = 88
T_FRAMES = 2584
T_PAD = 2592
F_BINS = 1025
F_PAD = 1152

BT = 432
BF = 128
NSUB = 3
BFG = NSUB * BF

def _edges():
    ratio = 1.059463094
    lowest = 27.5
    hz = [lowest * ratio ** i for i in range(89)]
    fe = np.array([(x + y) / 2 for x, y in zip([lowest / ratio] + hz, hz)],
                  dtype=np.float64)
    te = np.arange(0.0, 60.0, 0.01)
    return te.astype(np.float32), fe.astype(np.float32)

_TE_F32, _FE_F32 = _edges()
TE_LEN = 6000
ED_PAD = 6144
_EDGE_TABLE = np.zeros((ED_PAD,), np.float32)
_EDGE_TABLE[:TE_LEN] = _TE_F32
_EDGE_TABLE[TE_LEN:TE_LEN + 89] = _FE_F32

_TE0 = np.float32(_TE_F32[0])
_TEL = np.float32(_TE_F32[-1])
_FE0 = np.float32(_FE_F32[0])
_FEL = np.float32(_FE_F32[-1])
_LOG2A = np.float64(np.log2(np.float64(27.5 * (1.0 + 1.059463094) / (2.0 * 1.059463094))))

NBINS = NT * NF
HPAD = 527936

def _build_w():
    n64 = np.arange(N_FFT, dtype=np.float64)
    win = (0.5 - 0.5 * np.cos(2.0 * np.pi * n64 / N_FFT)).astype(np.float32)
    dwin = ((np.roll(win, -1) - np.roll(win, 1))
            * np.float32(0.5)).astype(np.float32)
    wtimes = (n64 + 0.5 - N_FFT // 2).astype(np.float32)
    twin = (win * wtimes).astype(np.float32)

    k = np.arange(F_PAD, dtype=np.float64)[None, :]
    ang = 2.0 * np.pi * n64[:, None] * k / N_FFT
    cos = np.cos(ang)
    msin = -np.sin(ang)
    cos[:, F_BINS:] = 0.0
    msin[:, F_BINS:] = 0.0

    w = np.empty((N_FFT, 9, 6, BF), np.float64)
    for g, wv in enumerate((win, dwin, twin)):
        w[:, :, 2 * g, :] = wv[:, None, None] * cos.reshape(N_FFT, 9, BF)
        w[:, :, 2 * g + 1, :] = wv[:, None, None] * msin.reshape(N_FFT, 9, BF)
    w32 = w.reshape(N_FFT, 9 * 6 * BF).astype(np.float32)
    import ml_dtypes
    whi = w32.astype(ml_dtypes.bfloat16)
    wlo = (w32 - whi.astype(np.float32)).astype(ml_dtypes.bfloat16)
    return w32, np.asarray(whi), np.asarray(wlo)

_W32_NP, _WHI_NP, _WLO_NP = _build_w()

_HOP_SR = np.float32(512.0 / 22050.0)
_SR_NFFT = np.float32(11025.0 / 1024.0)
_FREQ_C = np.float32(0.5 * 22050.0 / np.pi)
_SR_F = np.float32(22050.0)


def _tc_body(c0, c1, c2, c3, whi, wlo, t_out, f_out, w_out):
    i = pl.program_id(1)
    j = pl.program_id(0)

    def split(x):
        h = x.astype(jnp.bfloat16)
        return h, (x - h.astype(jnp.float32)).astype(jnp.bfloat16)

    ch, cl = zip(*(split(c[...]) for c in (c0, c1, c2, c3)))

    ts, fs, ws = [], [], []
    for sub in range(NSUB):
        acc = jnp.zeros((BT, 6 * BF), jnp.float32)
        for kk in range(4):
            wh = whi[pl.ds(kk * 512, 512), pl.ds(sub * 6 * BF, 6 * BF)]
            wl = wlo[pl.ds(kk * 512, 512), pl.ds(sub * 6 * BF, 6 * BF)]
            d = lambda x, y: jnp.dot(x, y, preferred_element_type=jnp.float32)
            acc += d(ch[kk], wh) + (d(ch[kk], wl) + d(cl[kk], wh))

        re_h = acc[:, 0 * BF:1 * BF]
        im_h = acc[:, 1 * BF:2 * BF]
        re_dh = acc[:, 2 * BF:3 * BF]
        im_dh = acc[:, 3 * BF:4 * BF]
        re_th = acc[:, 4 * BF:5 * BF]
        im_th = acc[:, 5 * BF:6 * BF]

        power = re_h * re_h + im_h * im_h
        mags = jnp.sqrt(power)
        bad = power < np.float32(REF_POWER)

        freq_corr = -((im_dh * re_h - re_dh * im_h) / power) * _FREQ_C
        time_corr = ((re_th * re_h + im_th * im_h) / power) / _SR_F

        rows = i * BT + lax.broadcasted_iota(jnp.int32, (BT, 1), 0)
        ft = rows.astype(jnp.float32) * _HOP_SR
        cols = (j * BFG + sub * BF
                + lax.broadcasted_iota(jnp.int32, (1, BF), 1))
        bf = cols.astype(jnp.float32) * _SR_NFFT

        times = jnp.where(bad, jnp.broadcast_to(ft, power.shape), ft + time_corr)
        freqs = jnp.where(bad, jnp.broadcast_to(bf, power.shape), bf + freq_corr)
        valid = ((times >= _TE0) & (times <= _TEL)
                 & (freqs >= _FE0) & (freqs <= _FEL))
        wgt = jnp.where(valid, mags, np.float32(0.0))
        ts.append(times)
        fs.append(freqs)
        ws.append(wgt)

    t_out[...] = jnp.concatenate(ts, axis=1)
    f_out[...] = jnp.concatenate(fs, axis=1)
    w_out[...] = jnp.concatenate(ws, axis=1)


def _run_tc(c0, c1, c2, c3, whi, wlo):
    spec_c = pl.BlockSpec((BT, 512), lambda j, i: (i, 0))
    spec_w = pl.BlockSpec((N_FFT, 6 * BFG), lambda j, i: (0, j))
    spec_o = pl.BlockSpec((BT, BFG), lambda j, i: (i, j))
    out_sh = jax.ShapeDtypeStruct((T_PAD, F_PAD), jnp.float32)
    return pl.pallas_call(
        _tc_body,
        grid=(F_PAD // BFG, T_PAD // BT),
        in_specs=[spec_c, spec_c, spec_c, spec_c, spec_w, spec_w],
        out_specs=[spec_o, spec_o, spec_o],
        out_shape=[out_sh, out_sh, out_sh],
    )(c0, c1, c2, c3, whi, wlo)


N_ELEM = T_PAD * F_PAD
ELEM_PT = N_ELEM // 32
CHUNK = 1152
CHUNK_R = CHUNK // 128
N_CHUNK = ELEM_PT // CHUNK
N_PAIR = N_CHUNK // 2


def _bin16(t16, f16, ed_v):
    c0 = jnp.clip(t16 * np.float32(100.0), np.float32(-10.0),
                  np.float32(6100.0)).astype(jnp.int32)
    bt = jnp.clip(c0 - 1, 0, TE_LEN - 3)
    e0 = plsc.load_gather(ed_v, [bt])
    e1 = plsc.load_gather(ed_v, [bt + 1])
    e2 = plsc.load_gather(ed_v, [bt + 2])
    cnt = ((t16 >= e0).astype(jnp.int32) + (t16 >= e1).astype(jnp.int32)
           + (t16 >= e2).astype(jnp.int32))
    ti = jnp.clip(bt + cnt - 1, 0, NT - 1)
    bits = plsc.bitcast(f16, jnp.int32)
    ex = (lax.shift_right_logical(bits, 23) & 255) - 127
    mant = (bits & 0x7FFFFF).astype(jnp.float32) * np.float32(2.0 ** -23)
    l2 = ex.astype(jnp.float32) + mant + mant * (np.float32(1.0) - mant) * np.float32(0.343)
    jf = (l2 - np.float32(_LOG2A)) * np.float32(12.0)
    j0 = jnp.clip(jf, np.float32(-10.0), np.float32(200.0)).astype(jnp.int32)
    j0 = jnp.where(f16 >= np.float32(1.0), j0, 0)
    bfq = jnp.clip(j0 - 1, 0, NF - 2) + TE_LEN
    g0 = plsc.load_gather(ed_v, [bfq])
    g1 = plsc.load_gather(ed_v, [bfq + 1])
    g2 = plsc.load_gather(ed_v, [bfq + 2])
    cf = ((f16 >= g0).astype(jnp.int32) + (f16 >= g1).astype(jnp.int32)
          + (f16 >= g2).astype(jnp.int32))
    fi = jnp.clip(bfq - TE_LEN + cf - 1, 0, NF - 1)
    return ti * NF + fi


def _sc_body(t_hbm, f_hbm, w_hbm, ed_hbm, z_hbm, out0_hbm, out1_hbm,
             ed_v, tv0, fv0, wv0, iv0, tv1, fv1, wv1, iv1, hist,
             lsem0, lsem1, ssem0, ssem1):
    c = lax.axis_index("c")
    s = lax.axis_index("s")
    wid = s * 2 + c
    pltpu.sync_copy(ed_hbm, ed_v)

    @pl.when(s == 0)
    def _():
        pltpu.sync_copy(z_hbm, hist)
    plsc.subcore_barrier()

    base = wid * ELEM_PT

    def loads(e0, tvb, fvb, wvb, sem):
        pltpu.async_copy(t_hbm.at[pl.ds(e0, CHUNK)], tvb, sem)
        pltpu.async_copy(f_hbm.at[pl.ds(e0, CHUNK)], fvb, sem)
        pltpu.async_copy(w_hbm.at[pl.ds(e0, CHUNK)], wvb, sem)

    def wait_loads(tvb, fvb, wvb, sem):
        pltpu.make_async_copy(t_hbm.at[pl.ds(0, CHUNK)], tvb, sem).wait()
        pltpu.make_async_copy(f_hbm.at[pl.ds(0, CHUNK)], fvb, sem).wait()
        pltpu.make_async_copy(w_hbm.at[pl.ds(0, CHUNK)], wvb, sem).wait()

    def compute(tvb, fvb, ivb):
        def row(r, carry):
            for v in range(8):
                t16 = tvb[pl.ds(r * 128 + v * 16, 16)]
                f16 = fvb[pl.ds(r * 128 + v * 16, 16)]
                ivb[r, pl.ds(v * 16, 16)] = _bin16(t16, f16, ed_v)
            return carry
        lax.fori_loop(0, CHUNK_R, row, 0)

    def scatter(wvb, ivb, sem):
        for r in range(CHUNK_R):
            pltpu.async_copy(wvb.at[pl.ds(r * 128, 128)],
                             hist.at[ivb.at[r]], sem, add=True)

    def wait_scatter(wvb, ivb, sem):
        for r in range(CHUNK_R):
            pltpu.make_async_copy(wvb.at[pl.ds(r * 128, 128)],
                                  hist.at[ivb.at[r]], sem).wait()

    loads(base, tv0, fv0, wv0, lsem0)
    loads(base + CHUNK, tv1, fv1, wv1, lsem1)

    def body(k, carry):
        e_a = base + 2 * k * CHUNK
        wait_loads(tv0, fv0, wv0, lsem0)
        compute(tv0, fv0, iv0)
        scatter(wv0, iv0, ssem0)
        wait_loads(tv1, fv1, wv1, lsem1)
        compute(tv1, fv1, iv1)
        scatter(wv1, iv1, ssem1)

        @pl.when(k < N_PAIR - 1)
        def _():
            wait_scatter(wv0, iv0, ssem0)
            loads(e_a + 2 * CHUNK, tv0, fv0, wv0, lsem0)
            wait_scatter(wv1, iv1, ssem1)
            loads(e_a + 3 * CHUNK, tv1, fv1, wv1, lsem1)

        @pl.when(k == N_PAIR - 1)
        def _():
            wait_scatter(wv0, iv0, ssem0)
            loads(base + (N_CHUNK - 1) * CHUNK, tv0, fv0, wv0, lsem0)
        return carry

    lax.fori_loop(0, N_PAIR, body, 0)
    wait_loads(tv0, fv0, wv0, lsem0)
    compute(tv0, fv0, iv0)
    scatter(wv0, iv0, ssem0)
    wait_scatter(wv0, iv0, ssem0)
    wait_scatter(wv1, iv1, ssem1)
    plsc.subcore_barrier()

    @pl.when((s == 0) & (c == 0))
    def _():
        pltpu.sync_copy(hist, out0_hbm)

    @pl.when((s == 0) & (c == 1))
    def _():
        pltpu.sync_copy(hist, out1_hbm)


@functools.lru_cache(maxsize=1)
def _get_sc_hist():
    mesh = plsc.VectorSubcoreMesh(core_axis_name="c", subcore_axis_name="s")
    return functools.partial(
        pl.kernel,
        mesh=mesh,
        compiler_params=pltpu.CompilerParams(needs_layout_passes=False),
        out_type=[jax.ShapeDtypeStruct((HPAD,), jnp.float32),
                  jax.ShapeDtypeStruct((HPAD,), jnp.float32)],
        scratch_types=[
            pltpu.VMEM((ED_PAD,), jnp.float32),
            pltpu.VMEM((CHUNK,), jnp.float32),
            pltpu.VMEM((CHUNK,), jnp.float32),
            pltpu.VMEM((CHUNK,), jnp.float32),
            pltpu.VMEM((CHUNK_R, 128), jnp.int32),
            pltpu.VMEM((CHUNK,), jnp.float32),
            pltpu.VMEM((CHUNK,), jnp.float32),
            pltpu.VMEM((CHUNK,), jnp.float32),
            pltpu.VMEM((CHUNK_R, 128), jnp.int32),
            pltpu.VMEM_SHARED((HPAD,), jnp.float32),
            pltpu.SemaphoreType.DMA,
            pltpu.SemaphoreType.DMA,
            pltpu.SemaphoreType.DMA,
            pltpu.SemaphoreType.DMA,
        ],
    )(_sc_body)


def kernel(waveform):
    pad = N_FFT // 2
    ypad = jnp.pad(waveform, (pad, pad))
    cgrid = ypad[: (T_FRAMES + 3) * HOP].reshape(T_FRAMES + 3, HOP)
    cs = [jnp.pad(cgrid[k:T_FRAMES + k], ((0, T_PAD - T_FRAMES), (0, 0)))
          for k in range(4)]

    whi = jnp.asarray(_WHI_NP)
    wlo = jnp.asarray(_WLO_NP)

    t, f, w = _run_tc(cs[0], cs[1], cs[2], cs[3], whi, wlo)

    t1 = t.reshape(N_ELEM)
    f1 = f.reshape(N_ELEM)
    w1 = w.reshape(N_ELEM)
    ed = jnp.asarray(_EDGE_TABLE)
    z = jnp.zeros((HPAD,), jnp.float32)

    p0, p1 = _get_sc_hist()(t1, f1, w1, ed, z)
    return (p0[:NBINS] + p1[:NBINS]).reshape(NT, NF)


# speedup vs baseline: 1030.7775x; 1.0736x over previous
import functools

import numpy as np
import jax
import jax.numpy as jnp
from jax import lax
from jax.experimental import pallas as pl
from jax.experimental.pallas import tpu as pltpu
from jax.experimental.pallas import tpu_sc as plsc

SR = 22050
N_FFT = 2048
HOP = 512
REF_POWER = 1e-6

NT = 5999
NF = 88
T_FRAMES = 2584
T_PAD = 2592
F_BINS = 1025
F_PAD = 1152

BT = 432
BF = 128
NSUB = 3
BFG = NSUB * BF

def _edges():
    ratio = 1.059463094
    lowest = 27.5
    hz = [lowest * ratio ** i for i in range(89)]
    fe = np.array([(x + y) / 2 for x, y in zip([lowest / ratio] + hz, hz)],
                  dtype=np.float64)
    te = np.arange(0.0, 60.0, 0.01)
    return te.astype(np.float32), fe.astype(np.float32)

_TE_F32, _FE_F32 = _edges()
TE_LEN = 6000
ED_PAD = 6144
_EDGE_TABLE = np.zeros((ED_PAD,), np.float32)
_EDGE_TABLE[:TE_LEN] = _TE_F32
_EDGE_TABLE[TE_LEN:TE_LEN + 89] = _FE_F32

_TE0 = np.float32(_TE_F32[0])
_TEL = np.float32(_TE_F32[-1])
_FE0 = np.float32(_FE_F32[0])
_FEL = np.float32(_FE_F32[-1])
_LOG2A = np.float64(np.log2(np.float64(27.5 * (1.0 + 1.059463094) / (2.0 * 1.059463094))))

NBINS = NT * NF
HPAD = 527936

def _build_w():
    n64 = np.arange(N_FFT, dtype=np.float64)
    win = (0.5 - 0.5 * np.cos(2.0 * np.pi * n64 / N_FFT)).astype(np.float32)
    dwin = ((np.roll(win, -1) - np.roll(win, 1))
            * np.float32(0.5)).astype(np.float32)
    wtimes = (n64 + 0.5 - N_FFT // 2).astype(np.float32)
    twin = (win * wtimes).astype(np.float32)

    k = np.arange(F_PAD, dtype=np.float64)[None, :]
    ang = 2.0 * np.pi * n64[:, None] * k / N_FFT
    cos = np.cos(ang)
    msin = -np.sin(ang)
    cos[:, F_BINS:] = 0.0
    msin[:, F_BINS:] = 0.0

    w = np.empty((N_FFT, 9, 6, BF), np.float64)
    for g, wv in enumerate((win, dwin, twin)):
        w[:, :, 2 * g, :] = wv[:, None, None] * cos.reshape(N_FFT, 9, BF)
        w[:, :, 2 * g + 1, :] = wv[:, None, None] * msin.reshape(N_FFT, 9, BF)
    w32 = w.reshape(N_FFT, 9 * 6 * BF).astype(np.float32)
    import ml_dtypes
    whi = w32.astype(ml_dtypes.bfloat16)
    wlo = (w32 - whi.astype(np.float32)).astype(ml_dtypes.bfloat16)
    return w32, np.asarray(whi), np.asarray(wlo)

_W32_NP, _WHI_NP, _WLO_NP = _build_w()

_HOP_SR = np.float32(512.0 / 22050.0)
_SR_NFFT = np.float32(11025.0 / 1024.0)
_FREQ_C = np.float32(0.5 * 22050.0 / np.pi)
_SR_F = np.float32(22050.0)


def _tc_body(c0, c1, c2, c3, whi, wlo, t_out, f_out, w_out, *, t0, bt):
    i = pl.program_id(1)
    j = pl.program_id(0)

    def split(x):
        h = x.astype(jnp.bfloat16)
        return h, (x - h.astype(jnp.float32)).astype(jnp.bfloat16)

    ch, cl = zip(*(split(c[...]) for c in (c0, c1, c2, c3)))

    ts, fs, ws = [], [], []
    for sub in range(NSUB):
        acc = jnp.zeros((bt, 6 * BF), jnp.float32)
        for kk in range(4):
            wh = whi[pl.ds(kk * 512, 512), pl.ds(sub * 6 * BF, 6 * BF)]
            wl = wlo[pl.ds(kk * 512, 512), pl.ds(sub * 6 * BF, 6 * BF)]
            d = lambda x, y: jnp.dot(x, y, preferred_element_type=jnp.float32)
            acc += d(ch[kk], wh) + (d(ch[kk], wl) + d(cl[kk], wh))

        re_h = acc[:, 0 * BF:1 * BF]
        im_h = acc[:, 1 * BF:2 * BF]
        re_dh = acc[:, 2 * BF:3 * BF]
        im_dh = acc[:, 3 * BF:4 * BF]
        re_th = acc[:, 4 * BF:5 * BF]
        im_th = acc[:, 5 * BF:6 * BF]

        power = re_h * re_h + im_h * im_h
        mags = jnp.sqrt(power)
        bad = power < np.float32(REF_POWER)

        freq_corr = -((im_dh * re_h - re_dh * im_h) / power) * _FREQ_C
        time_corr = ((re_th * re_h + im_th * im_h) / power) / _SR_F

        rows = t0 + i * bt + lax.broadcasted_iota(jnp.int32, (bt, 1), 0)
        ft = rows.astype(jnp.float32) * _HOP_SR
        cols = (j * BFG + sub * BF
                + lax.broadcasted_iota(jnp.int32, (1, BF), 1))
        bf = cols.astype(jnp.float32) * _SR_NFFT

        times = jnp.where(bad, jnp.broadcast_to(ft, power.shape), ft + time_corr)
        freqs = jnp.where(bad, jnp.broadcast_to(bf, power.shape), bf + freq_corr)
        valid = ((times >= _TE0) & (times <= _TEL)
                 & (freqs >= _FE0) & (freqs <= _FEL))
        wgt = jnp.where(valid, mags, np.float32(0.0))
        ts.append(times)
        fs.append(freqs)
        ws.append(wgt)

    t_out[...] = jnp.concatenate(ts, axis=1)
    f_out[...] = jnp.concatenate(fs, axis=1)
    w_out[...] = jnp.concatenate(ws, axis=1)


def _run_tc(c0, c1, c2, c3, whi, wlo, t0, t_len, bt):
    spec_c = pl.BlockSpec((bt, 512), lambda j, i: (i, 0))
    spec_w = pl.BlockSpec((N_FFT, 6 * BFG), lambda j, i: (0, j))
    spec_o = pl.BlockSpec((bt, BFG), lambda j, i: (i, j))
    out_sh = jax.ShapeDtypeStruct((t_len, F_PAD), jnp.float32)
    body = functools.partial(_tc_body, t0=t0, bt=bt)
    return pl.pallas_call(
        body,
        grid=(F_PAD // BFG, t_len // bt),
        in_specs=[spec_c, spec_c, spec_c, spec_c, spec_w, spec_w],
        out_specs=[spec_o, spec_o, spec_o],
        out_shape=[out_sh, out_sh, out_sh],
    )(c0, c1, c2, c3, whi, wlo)


N_ELEM = T_PAD * F_PAD
ELEM_PT = N_ELEM // 32
CHUNK = 1152
CHUNK_R = CHUNK // 128
N_CHUNK = ELEM_PT // CHUNK
N_PAIR = N_CHUNK // 2


def _bin16(t16, f16, ed_v):
    c0 = jnp.clip(t16 * np.float32(100.0), np.float32(-10.0),
                  np.float32(6100.0)).astype(jnp.int32)
    bt = jnp.clip(c0 - 1, 0, TE_LEN - 3)
    e0 = plsc.load_gather(ed_v, [bt])
    e1 = plsc.load_gather(ed_v, [bt + 1])
    e2 = plsc.load_gather(ed_v, [bt + 2])
    cnt = ((t16 >= e0).astype(jnp.int32) + (t16 >= e1).astype(jnp.int32)
           + (t16 >= e2).astype(jnp.int32))
    ti = jnp.clip(bt + cnt - 1, 0, NT - 1)
    bits = plsc.bitcast(f16, jnp.int32)
    ex = (lax.shift_right_logical(bits, 23) & 255) - 127
    mant = (bits & 0x7FFFFF).astype(jnp.float32) * np.float32(2.0 ** -23)
    l2 = ex.astype(jnp.float32) + mant + mant * (np.float32(1.0) - mant) * np.float32(0.343)
    jf = (l2 - np.float32(_LOG2A)) * np.float32(12.0)
    j0 = jnp.clip(jf, np.float32(-10.0), np.float32(200.0)).astype(jnp.int32)
    j0 = jnp.where(f16 >= np.float32(1.0), j0, 0)
    bfq = jnp.clip(j0 - 1, 0, NF - 2) + TE_LEN
    g0 = plsc.load_gather(ed_v, [bfq])
    g1 = plsc.load_gather(ed_v, [bfq + 1])
    g2 = plsc.load_gather(ed_v, [bfq + 2])
    cf = ((f16 >= g0).astype(jnp.int32) + (f16 >= g1).astype(jnp.int32)
          + (f16 >= g2).astype(jnp.int32))
    fi = jnp.clip(bfq - TE_LEN + cf - 1, 0, NF - 1)
    return ti * NF + fi


def _make_sc_body(elem_pt, n_chunk):
  n_pair = n_chunk // 2
  tail = n_chunk % 2

  def _sc_body(t_hbm, f_hbm, w_hbm, ed_hbm, z_hbm, out0_hbm, out1_hbm,
               ed_v, tv0, fv0, wv0, iv0, tv1, fv1, wv1, iv1, hist,
               lsem0, lsem1, ssem0, ssem1):
    c = lax.axis_index("c")
    s = lax.axis_index("s")
    wid = s * 2 + c
    pltpu.sync_copy(ed_hbm, ed_v)

    @pl.when(s == 0)
    def _():
        pltpu.sync_copy(z_hbm, hist)
    plsc.subcore_barrier()

    base = wid * elem_pt

    def loads(e0, tvb, fvb, wvb, sem):
        pltpu.async_copy(t_hbm.at[pl.ds(e0, CHUNK)], tvb, sem)
        pltpu.async_copy(f_hbm.at[pl.ds(e0, CHUNK)], fvb, sem)
        pltpu.async_copy(w_hbm.at[pl.ds(e0, CHUNK)], wvb, sem)

    def wait_loads(tvb, fvb, wvb, sem):
        pltpu.make_async_copy(t_hbm.at[pl.ds(0, CHUNK)], tvb, sem).wait()
        pltpu.make_async_copy(f_hbm.at[pl.ds(0, CHUNK)], fvb, sem).wait()
        pltpu.make_async_copy(w_hbm.at[pl.ds(0, CHUNK)], wvb, sem).wait()

    def compute(tvb, fvb, ivb):
        def row(r, carry):
            for v in range(8):
                t16 = tvb[pl.ds(r * 128 + v * 16, 16)]
                f16 = fvb[pl.ds(r * 128 + v * 16, 16)]
                ivb[r, pl.ds(v * 16, 16)] = _bin16(t16, f16, ed_v)
            return carry
        lax.fori_loop(0, CHUNK_R, row, 0)

    def scatter(wvb, ivb, sem):
        for r in range(CHUNK_R):
            pltpu.async_copy(wvb.at[pl.ds(r * 128, 128)],
                             hist.at[ivb.at[r]], sem, add=True)

    def wait_scatter(wvb, ivb, sem):
        for r in range(CHUNK_R):
            pltpu.make_async_copy(wvb.at[pl.ds(r * 128, 128)],
                                  hist.at[ivb.at[r]], sem).wait()

    loads(base, tv0, fv0, wv0, lsem0)
    loads(base + CHUNK, tv1, fv1, wv1, lsem1)

    def body(k, carry):
        e_a = base + 2 * k * CHUNK
        wait_loads(tv0, fv0, wv0, lsem0)
        compute(tv0, fv0, iv0)
        scatter(wv0, iv0, ssem0)
        wait_loads(tv1, fv1, wv1, lsem1)
        compute(tv1, fv1, iv1)
        scatter(wv1, iv1, ssem1)

        @pl.when(k < n_pair - 1)
        def _():
            wait_scatter(wv0, iv0, ssem0)
            loads(e_a + 2 * CHUNK, tv0, fv0, wv0, lsem0)
            wait_scatter(wv1, iv1, ssem1)
            loads(e_a + 3 * CHUNK, tv1, fv1, wv1, lsem1)

        if tail:
            @pl.when(k == n_pair - 1)
            def _():
                wait_scatter(wv0, iv0, ssem0)
                loads(base + (n_chunk - 1) * CHUNK, tv0, fv0, wv0, lsem0)
        return carry

    lax.fori_loop(0, n_pair, body, 0)
    if tail:
        wait_loads(tv0, fv0, wv0, lsem0)
        compute(tv0, fv0, iv0)
        scatter(wv0, iv0, ssem0)
    wait_scatter(wv0, iv0, ssem0)
    wait_scatter(wv1, iv1, ssem1)
    plsc.subcore_barrier()

    @pl.when((s == 0) & (c == 0))
    def _():
        pltpu.sync_copy(hist, out0_hbm)

    @pl.when((s == 0) & (c == 1))
    def _():
        pltpu.sync_copy(hist, out1_hbm)

  return _sc_body


@functools.lru_cache(maxsize=4)
def _get_sc_hist(elem_pt, n_chunk):
    mesh = plsc.VectorSubcoreMesh(core_axis_name="c", subcore_axis_name="s")
    return functools.partial(
        pl.kernel,
        mesh=mesh,
        compiler_params=pltpu.CompilerParams(needs_layout_passes=False),
        out_type=[jax.ShapeDtypeStruct((HPAD,), jnp.float32),
                  jax.ShapeDtypeStruct((HPAD,), jnp.float32)],
        scratch_types=[
            pltpu.VMEM((ED_PAD,), jnp.float32),
            pltpu.VMEM((CHUNK,), jnp.float32),
            pltpu.VMEM((CHUNK,), jnp.float32),
            pltpu.VMEM((CHUNK,), jnp.float32),
            pltpu.VMEM((CHUNK_R, 128), jnp.int32),
            pltpu.VMEM((CHUNK,), jnp.float32),
            pltpu.VMEM((CHUNK,), jnp.float32),
            pltpu.VMEM((CHUNK,), jnp.float32),
            pltpu.VMEM((CHUNK_R, 128), jnp.int32),
            pltpu.VMEM_SHARED((HPAD,), jnp.float32),
            pltpu.SemaphoreType.DMA,
            pltpu.SemaphoreType.DMA,
            pltpu.SemaphoreType.DMA,
            pltpu.SemaphoreType.DMA,
        ],
    )(_make_sc_body(elem_pt, n_chunk))


def kernel(waveform):
    pad = N_FFT // 2
    ypad = jnp.pad(waveform, (pad, pad))
    cgrid = ypad[: (T_FRAMES + 3) * HOP].reshape(T_FRAMES + 3, HOP)
    cs = [jnp.pad(cgrid[k:T_FRAMES + k], ((0, T_PAD - T_FRAMES), (0, 0)))
          for k in range(4)]

    whi = jnp.asarray(_WHI_NP)
    wlo = jnp.asarray(_WLO_NP)
    ed = jnp.asarray(_EDGE_TABLE)
    z = jnp.zeros((HPAD,), jnp.float32)

    TA, TB = 1280, 1312
    BTA, BTB = 256, 328
    parts = []
    for (lo, t_len, bt) in ((0, TA, BTA), (TA, TB, BTB)):
        csx = [x[lo:lo + t_len] for x in cs]
        t, f, w = _run_tc(csx[0], csx[1], csx[2], csx[3], whi, wlo,
                          lo, t_len, bt)
        n_elem = t_len * F_PAD
        elem_pt = n_elem // 32
        p0, p1 = _get_sc_hist(elem_pt, elem_pt // CHUNK)(
            t.reshape(n_elem), f.reshape(n_elem), w.reshape(n_elem), ed, z)
        parts.append(p0)
        parts.append(p1)

    acc = (parts[0][:NBINS] + parts[1][:NBINS]) + (parts[2][:NBINS] + parts[3][:NBINS])
    return acc.reshape(NT, NF)


# speedup vs baseline: 1068.9447x; 1.0370x over previous
import functools

import numpy as np
import jax
import jax.numpy as jnp
from jax import lax
from jax.experimental import pallas as pl
from jax.experimental.pallas import tpu as pltpu
from jax.experimental.pallas import tpu_sc as plsc

SR = 22050
N_FFT = 2048
HOP = 512
REF_POWER = 1e-6

NT = 5999
NF = 88
T_FRAMES = 2584
T_PAD = 2592
F_BINS = 1025
F_PAD = 1152

BT = 432
BF = 128
NSUB = 3
BFG = NSUB * BF

def _edges():
    ratio = 1.059463094
    lowest = 27.5
    hz = [lowest * ratio ** i for i in range(89)]
    fe = np.array([(x + y) / 2 for x, y in zip([lowest / ratio] + hz, hz)],
                  dtype=np.float64)
    te = np.arange(0.0, 60.0, 0.01)
    return te.astype(np.float32), fe.astype(np.float32)

_TE_F32, _FE_F32 = _edges()
TE_LEN = 6000
ED_PAD = 6144
_EDGE_TABLE = np.zeros((ED_PAD,), np.float32)
_EDGE_TABLE[:TE_LEN] = _TE_F32
_EDGE_TABLE[TE_LEN:TE_LEN + 89] = _FE_F32

_TE0 = np.float32(_TE_F32[0])
_TEL = np.float32(_TE_F32[-1])
_FE0 = np.float32(_FE_F32[0])
_FEL = np.float32(_FE_F32[-1])
_LOG2A = np.float64(np.log2(np.float64(27.5 * (1.0 + 1.059463094) / (2.0 * 1.059463094))))

NBINS = NT * NF
HPAD = 527936

def _build_w():
    n64 = np.arange(N_FFT, dtype=np.float64)
    win = (0.5 - 0.5 * np.cos(2.0 * np.pi * n64 / N_FFT)).astype(np.float32)
    dwin = ((np.roll(win, -1) - np.roll(win, 1))
            * np.float32(0.5)).astype(np.float32)
    wtimes = (n64 + 0.5 - N_FFT // 2).astype(np.float32)
    twin = (win * wtimes).astype(np.float32)

    k = np.arange(F_PAD, dtype=np.float64)[None, :]
    ang = 2.0 * np.pi * n64[:, None] * k / N_FFT
    cos = np.cos(ang)
    msin = -np.sin(ang)
    cos[:, F_BINS:] = 0.0
    msin[:, F_BINS:] = 0.0

    w = np.empty((N_FFT, 9, 6, BF), np.float64)
    for g, wv in enumerate((win, dwin, twin)):
        w[:, :, 2 * g, :] = wv[:, None, None] * cos.reshape(N_FFT, 9, BF)
        w[:, :, 2 * g + 1, :] = wv[:, None, None] * msin.reshape(N_FFT, 9, BF)
    w32 = w.reshape(N_FFT, 9 * 6 * BF).astype(np.float32)
    import ml_dtypes
    whi = w32.astype(ml_dtypes.bfloat16)
    wlo = (w32 - whi.astype(np.float32)).astype(ml_dtypes.bfloat16)
    return w32, np.asarray(whi), np.asarray(wlo)

_W32_NP, _WHI_NP, _WLO_NP = _build_w()

_HOP_SR = np.float32(512.0 / 22050.0)
_SR_NFFT = np.float32(11025.0 / 1024.0)
_FREQ_C = np.float32(0.5 * 22050.0 / np.pi)
_SR_F = np.float32(22050.0)


def _tc_body(c0, c1, c2, c3, whi, wlo, t_out, f_out, w_out, *, t0, bt):
    i = pl.program_id(1)
    j = pl.program_id(0)

    def split(x):
        h = x.astype(jnp.bfloat16)
        return h, (x - h.astype(jnp.float32)).astype(jnp.bfloat16)

    ch, cl = zip(*(split(c[...]) for c in (c0, c1, c2, c3)))

    ts, fs, ws = [], [], []
    for sub in range(NSUB):
        acc = jnp.zeros((bt, 6 * BF), jnp.float32)
        for kk in range(4):
            wh = whi[pl.ds(kk * 512, 512), pl.ds(sub * 6 * BF, 6 * BF)]
            wl = wlo[pl.ds(kk * 512, 512), pl.ds(sub * 6 * BF, 6 * BF)]
            d = lambda x, y: jnp.dot(x, y, preferred_element_type=jnp.float32)
            acc += d(ch[kk], wh) + (d(ch[kk], wl) + d(cl[kk], wh))

        re_h = acc[:, 0 * BF:1 * BF]
        im_h = acc[:, 1 * BF:2 * BF]
        re_dh = acc[:, 2 * BF:3 * BF]
        im_dh = acc[:, 3 * BF:4 * BF]
        re_th = acc[:, 4 * BF:5 * BF]
        im_th = acc[:, 5 * BF:6 * BF]

        power = re_h * re_h + im_h * im_h
        mags = jnp.sqrt(power)
        bad = power < np.float32(REF_POWER)

        freq_corr = -((im_dh * re_h - re_dh * im_h) / power) * _FREQ_C
        time_corr = ((re_th * re_h + im_th * im_h) / power) / _SR_F

        rows = t0 + i * bt + lax.broadcasted_iota(jnp.int32, (bt, 1), 0)
        ft = rows.astype(jnp.float32) * _HOP_SR
        cols = (j * BFG + sub * BF
                + lax.broadcasted_iota(jnp.int32, (1, BF), 1))
        bf = cols.astype(jnp.float32) * _SR_NFFT

        times = jnp.where(bad, jnp.broadcast_to(ft, power.shape), ft + time_corr)
        freqs = jnp.where(bad, jnp.broadcast_to(bf, power.shape), bf + freq_corr)
        valid = ((times >= _TE0) & (times <= _TEL)
                 & (freqs >= _FE0) & (freqs <= _FEL))
        wgt = jnp.where(valid, mags, np.float32(0.0))
        ts.append(times)
        fs.append(freqs)
        ws.append(wgt)

    t_out[...] = jnp.concatenate(ts, axis=1)
    f_out[...] = jnp.concatenate(fs, axis=1)
    w_out[...] = jnp.concatenate(ws, axis=1)


def _run_tc(c0, c1, c2, c3, whi, wlo, t0, t_len, bt):
    spec_c = pl.BlockSpec((bt, 512), lambda j, i: (i, 0))
    spec_w = pl.BlockSpec((N_FFT, 6 * BFG), lambda j, i: (0, j))
    spec_o = pl.BlockSpec((bt, BFG), lambda j, i: (i, j))
    out_sh = jax.ShapeDtypeStruct((t_len, F_PAD), jnp.float32)
    body = functools.partial(_tc_body, t0=t0, bt=bt)
    return pl.pallas_call(
        body,
        grid=(F_PAD // BFG, t_len // bt),
        in_specs=[spec_c, spec_c, spec_c, spec_c, spec_w, spec_w],
        out_specs=[spec_o, spec_o, spec_o],
        out_shape=[out_sh, out_sh, out_sh],
    )(c0, c1, c2, c3, whi, wlo)


N_ELEM = T_PAD * F_PAD
ELEM_PT = N_ELEM // 32
CHUNK = 1152
CHUNK_R = CHUNK // 128
N_CHUNK = ELEM_PT // CHUNK
N_PAIR = N_CHUNK // 2


def _bin16(t16, f16, ed_v):
    c0 = jnp.clip(t16 * np.float32(100.0), np.float32(-10.0),
                  np.float32(6100.0)).astype(jnp.int32)
    bt = jnp.clip(c0 - 1, 0, TE_LEN - 3)
    e0 = plsc.load_gather(ed_v, [bt])
    e1 = plsc.load_gather(ed_v, [bt + 1])
    e2 = plsc.load_gather(ed_v, [bt + 2])
    cnt = ((t16 >= e0).astype(jnp.int32) + (t16 >= e1).astype(jnp.int32)
           + (t16 >= e2).astype(jnp.int32))
    ti = jnp.clip(bt + cnt - 1, 0, NT - 1)
    bits = plsc.bitcast(f16, jnp.int32)
    ex = (lax.shift_right_logical(bits, 23) & 255) - 127
    mant = (bits & 0x7FFFFF).astype(jnp.float32) * np.float32(2.0 ** -23)
    l2 = ex.astype(jnp.float32) + mant + mant * (np.float32(1.0) - mant) * np.float32(0.343)
    jf = (l2 - np.float32(_LOG2A)) * np.float32(12.0)
    j0 = jnp.clip(jf, np.float32(-10.0), np.float32(200.0)).astype(jnp.int32)
    j0 = jnp.where(f16 >= np.float32(1.0), j0, 0)
    bfq = jnp.clip(j0 - 1, 0, NF - 2) + TE_LEN
    g0 = plsc.load_gather(ed_v, [bfq])
    g1 = plsc.load_gather(ed_v, [bfq + 1])
    g2 = plsc.load_gather(ed_v, [bfq + 2])
    cf = ((f16 >= g0).astype(jnp.int32) + (f16 >= g1).astype(jnp.int32)
          + (f16 >= g2).astype(jnp.int32))
    fi = jnp.clip(bfq - TE_LEN + cf - 1, 0, NF - 1)
    return ti * NF + fi


def _make_sc_body(elem_pt, n_chunk):
  n_pair = n_chunk // 2
  tail = n_chunk % 2

  def _sc_body(t_hbm, f_hbm, w_hbm, ed_hbm, z_hbm, out0_hbm, out1_hbm,
               ed_v, tv0, fv0, wv0, iv0, tv1, fv1, wv1, iv1, hist,
               lsem0, lsem1, ssem0, ssem1):
    c = lax.axis_index("c")
    s = lax.axis_index("s")
    wid = s * 2 + c
    pltpu.sync_copy(ed_hbm, ed_v)

    @pl.when(s == 0)
    def _():
        pltpu.sync_copy(z_hbm, hist)
    plsc.subcore_barrier()

    base = wid * elem_pt

    def loads(e0, tvb, fvb, wvb, sem):
        pltpu.async_copy(t_hbm.at[pl.ds(e0, CHUNK)], tvb, sem)
        pltpu.async_copy(f_hbm.at[pl.ds(e0, CHUNK)], fvb, sem)
        pltpu.async_copy(w_hbm.at[pl.ds(e0, CHUNK)], wvb, sem)

    def wait_loads(tvb, fvb, wvb, sem):
        pltpu.make_async_copy(t_hbm.at[pl.ds(0, CHUNK)], tvb, sem).wait()
        pltpu.make_async_copy(f_hbm.at[pl.ds(0, CHUNK)], fvb, sem).wait()
        pltpu.make_async_copy(w_hbm.at[pl.ds(0, CHUNK)], wvb, sem).wait()

    def compute(tvb, fvb, ivb):
        def row(r, carry):
            for v in range(8):
                t16 = tvb[pl.ds(r * 128 + v * 16, 16)]
                f16 = fvb[pl.ds(r * 128 + v * 16, 16)]
                ivb[r, pl.ds(v * 16, 16)] = _bin16(t16, f16, ed_v)
            return carry
        lax.fori_loop(0, CHUNK_R, row, 0)

    def scatter(wvb, ivb, sem):
        for r in range(CHUNK_R):
            pltpu.async_copy(wvb.at[pl.ds(r * 128, 128)],
                             hist.at[ivb.at[r]], sem, add=True)

    def wait_scatter(wvb, ivb, sem):
        for r in range(CHUNK_R):
            pltpu.make_async_copy(wvb.at[pl.ds(r * 128, 128)],
                                  hist.at[ivb.at[r]], sem).wait()

    loads(base, tv0, fv0, wv0, lsem0)
    loads(base + CHUNK, tv1, fv1, wv1, lsem1)

    def body(k, carry):
        e_a = base + 2 * k * CHUNK
        wait_loads(tv0, fv0, wv0, lsem0)
        compute(tv0, fv0, iv0)
        scatter(wv0, iv0, ssem0)
        wait_loads(tv1, fv1, wv1, lsem1)
        compute(tv1, fv1, iv1)
        scatter(wv1, iv1, ssem1)

        @pl.when(k < n_pair - 1)
        def _():
            wait_scatter(wv0, iv0, ssem0)
            loads(e_a + 2 * CHUNK, tv0, fv0, wv0, lsem0)
            wait_scatter(wv1, iv1, ssem1)
            loads(e_a + 3 * CHUNK, tv1, fv1, wv1, lsem1)

        if tail:
            @pl.when(k == n_pair - 1)
            def _():
                wait_scatter(wv0, iv0, ssem0)
                loads(base + (n_chunk - 1) * CHUNK, tv0, fv0, wv0, lsem0)
        return carry

    lax.fori_loop(0, n_pair, body, 0)
    if tail:
        wait_loads(tv0, fv0, wv0, lsem0)
        compute(tv0, fv0, iv0)
        scatter(wv0, iv0, ssem0)
    wait_scatter(wv0, iv0, ssem0)
    wait_scatter(wv1, iv1, ssem1)
    plsc.subcore_barrier()

    @pl.when((s == 0) & (c == 0))
    def _():
        pltpu.sync_copy(hist, out0_hbm)

    @pl.when((s == 0) & (c == 1))
    def _():
        pltpu.sync_copy(hist, out1_hbm)

  return _sc_body


@functools.lru_cache(maxsize=4)
def _get_sc_hist(elem_pt, n_chunk):
    mesh = plsc.VectorSubcoreMesh(core_axis_name="c", subcore_axis_name="s")
    return functools.partial(
        pl.kernel,
        mesh=mesh,
        compiler_params=pltpu.CompilerParams(needs_layout_passes=False),
        out_type=[jax.ShapeDtypeStruct((HPAD,), jnp.float32),
                  jax.ShapeDtypeStruct((HPAD,), jnp.float32)],
        scratch_types=[
            pltpu.VMEM((ED_PAD,), jnp.float32),
            pltpu.VMEM((CHUNK,), jnp.float32),
            pltpu.VMEM((CHUNK,), jnp.float32),
            pltpu.VMEM((CHUNK,), jnp.float32),
            pltpu.VMEM((CHUNK_R, 128), jnp.int32),
            pltpu.VMEM((CHUNK,), jnp.float32),
            pltpu.VMEM((CHUNK,), jnp.float32),
            pltpu.VMEM((CHUNK,), jnp.float32),
            pltpu.VMEM((CHUNK_R, 128), jnp.int32),
            pltpu.VMEM_SHARED((HPAD,), jnp.float32),
            pltpu.SemaphoreType.DMA,
            pltpu.SemaphoreType.DMA,
            pltpu.SemaphoreType.DMA,
            pltpu.SemaphoreType.DMA,
        ],
    )(_make_sc_body(elem_pt, n_chunk))


def kernel(waveform):
    pad = N_FFT // 2
    ypad = jnp.pad(waveform, (pad, pad))
    cgrid = ypad[: (T_FRAMES + 3) * HOP].reshape(T_FRAMES + 3, HOP)

    whi = jnp.asarray(_WHI_NP)
    wlo = jnp.asarray(_WLO_NP)
    ed = jnp.asarray(_EDGE_TABLE)
    z = jnp.zeros((HPAD,), jnp.float32)

    TA, TB = 1280, 1312
    BTA, BTB = 256, 328
    parts = []
    for (lo, t_len, bt) in ((0, TA, BTA), (TA, TB, BTB)):
        if lo + t_len <= T_FRAMES:
            csx = [cgrid[lo + k:lo + t_len + k] for k in range(4)]
        else:
            csx = [jnp.pad(cgrid[lo + k:T_FRAMES + k],
                           ((0, lo + t_len - T_FRAMES), (0, 0)))
                   for k in range(4)]
        t, f, w = _run_tc(csx[0], csx[1], csx[2], csx[3], whi, wlo,
                          lo, t_len, bt)
        n_elem = t_len * F_PAD
        elem_pt = n_elem // 32
        p0, p1 = _get_sc_hist(elem_pt, elem_pt // CHUNK)(
            t.reshape(n_elem), f.reshape(n_elem), w.reshape(n_elem), ed, z)
        parts.append(p0)
        parts.append(p1)

    acc = (parts[0][:NBINS] + parts[1][:NBINS]) + (parts[2][:NBINS] + parts[3][:NBINS])
    return acc.reshape(NT, NF)
